# Initial kernel scaffold; baseline (speedup 1.0000x reference)
import jax
import jax.numpy as jnp
from jax.experimental import pallas as pl


def kernel(locations, box_cls, box_regression, centerness, image_sizes):
    raise NotImplementedError("write your pallas kernel here")


# speedup vs baseline: 2.7867x; 2.7867x over previous
import jax
import jax.numpy as jnp
from jax import lax
from jax.experimental import pallas as pl
from jax.experimental.pallas import tpu as pltpu

_PRE_NMS_THRESH = 0.05
_PRE_NMS_TOP_N = 1000
_NMS_THRESH = 0.6
_POST_TOP_N = 100
_NEG = -1e9

_HW = 15200
_C = 80
_ROWS = (_HW * _C) // 128
_G0 = 1188
_G0P = 1192
_G1 = 149
_G1P = 152
_G2 = 19
_KP = 1024


def _t(x):
    return lax.dot_general(
        jnp.ones((1, 1), jnp.float32), x, (((1,), (1,)), ((), ())),
        precision=lax.Precision.HIGHEST,
        preferred_element_type=jnp.float32)


def _group_max8(x, rows):
    r = x.reshape(rows, 8, 128)
    g = r[:, 0, :]
    for s in range(1, 8):
        g = jnp.maximum(g, r[:, s, :])
    return g


def _fcos_kernel(cls_ref, ctr_ref, lr_ref, sz_ref,
                 boxes_ref, vals_ref, labels_ref,
                 score_scr, g_scr, g2_scr, det_scr, iou_scr, out_scr):
    f32 = jnp.float32
    i32 = jnp.int32

    sig_cls = jax.nn.sigmoid(cls_ref[0])
    sig_ctr = jax.nn.sigmoid(ctr_ref[0])
    cand = sig_cls > _PRE_NMS_THRESH
    pre_top = jnp.minimum(jnp.sum(cand.astype(i32)), _PRE_NMS_TOP_N)
    masked = jnp.where(cand, sig_cls * sig_ctr, _NEG)
    padded = jnp.concatenate(
        [masked, jnp.full((_G0 * 8 - _ROWS, 128), _NEG, f32)], axis=0)
    score_scr[...] = padded
    g0 = jnp.concatenate(
        [_group_max8(padded, _G0),
         jnp.full((_G0P - _G0, 128), _NEG, f32)], axis=0)
    g_scr[...] = g0
    g1 = jnp.concatenate(
        [_group_max8(g0, _G1),
         jnp.full((_G1P - _G1, 128), _NEG, f32)], axis=0)
    g2_scr[...] = g1
    g3_0 = _group_max8(g1, _G2)

    h = sz_ref[0, 0, 0]
    w = sz_ref[0, 0, 1]
    wm1 = w - 1.0
    hm1 = h - 1.0

    r19 = lax.broadcasted_iota(i32, (_G2, 128), 0)
    r8 = lax.broadcasted_iota(i32, (8, 128), 0)
    t_iota = (lax.broadcasted_iota(i32, (8, 128), 0) * 128 +
              lax.broadcasted_iota(i32, (8, 128), 1))
    big = jnp.int32(2 ** 30)

    def extract_body(k, g3):
        m1 = jnp.max(g3)
        a3 = jnp.min(jnp.where(g3 == m1, r19, big))
        g2t = g2_scr[pl.ds(a3 * 8, 8), :]
        a2 = a3 * 8 + jnp.min(jnp.where(g2t == m1, r8, big))
        g1t = g_scr[pl.ds(a2 * 8, 8), :]
        gr = a2 * 8 + jnp.min(jnp.where(g1t == m1, r8, big))
        tile = score_scr[pl.ds(gr * 8, 8), :]
        p = jnp.min(jnp.where(tile == m1, t_iota, big))
        idx = gr * 1024 + p

        ntile = jnp.where(t_iota == p, _NEG, tile)
        score_scr[pl.ds(gr * 8, 8), :] = ntile
        g_scr[pl.ds(gr, 1), :] = jnp.max(ntile, axis=0, keepdims=True)
        gslab = g_scr[pl.ds(a2 * 8, 8), :]
        g2_scr[pl.ds(a2, 1), :] = jnp.max(gslab, axis=0, keepdims=True)
        g2slab = g2_scr[pl.ds(a3 * 8, 8), :]
        nrow3 = jnp.max(g2slab, axis=0, keepdims=True)
        g3 = jnp.where(r19 == a3, jnp.broadcast_to(nrow3, (_G2, 128)), g3)

        li = idx // _C
        ci = (idx - li * _C + 1).astype(f32)
        lrv = lr_ref[0, pl.ds(li, 1), :]
        lx = lrv[0, 0]
        ly = lrv[0, 1]
        x1 = jnp.clip(lx - lrv[0, 2], 0.0, wm1)
        y1 = jnp.clip(ly - lrv[0, 3], 0.0, hm1)
        x2 = jnp.clip(lx + lrv[0, 4], 0.0, wm1)
        y2 = jnp.clip(ly + lrv[0, 5], 0.0, hm1)
        keep = ((m1 > -1e8) & (k < pre_top) &
                (x2 - x1 >= 0.0) & (y2 - y1 >= 0.0))
        kf = jnp.where(keep, 1.0, 0.0)
        scv = jnp.where(keep, m1, 0.0)
        row = jnp.concatenate(
            [x1.reshape(1, 1), y1.reshape(1, 1), x2.reshape(1, 1),
             y2.reshape(1, 1), scv.reshape(1, 1), ci.reshape(1, 1),
             kf.reshape(1, 1), jnp.zeros((1, 1), f32)], axis=1)
        det_scr[pl.ds(k, 1), :] = row
        return g3

    lax.fori_loop(0, _PRE_NMS_TOP_N, extract_body, g3_0)
    det_scr[_PRE_NMS_TOP_N:_KP, :] = jnp.zeros((_KP - _PRE_NMS_TOP_N, 8), f32)

    buf = det_scr[...]
    off = buf[:, 5:6] * 4096.0
    nx1c = buf[:, 0:1] + off
    ny1c = buf[:, 1:2] + off
    nx2c = buf[:, 2:3] + off
    ny2c = buf[:, 3:4] + off
    areac = (jnp.clip(nx2c - nx1c, 0.0, None) *
             jnp.clip(ny2c - ny1c, 0.0, None))
    nx1r = _t(nx1c)
    ny1r = _t(ny1c)
    nx2r = _t(nx2c)
    ny2r = _t(ny2c)
    arear = _t(areac)

    for s in range(_KP // 128):
        sl = slice(s * 128, (s + 1) * 128)
        bc = lambda q: jnp.broadcast_to(q[sl], (128, _KP))
        br = lambda q: jnp.broadcast_to(q, (128, _KP))
        ix1 = jnp.maximum(bc(nx1c), br(nx1r))
        iy1 = jnp.maximum(bc(ny1c), br(ny1r))
        ix2 = jnp.minimum(bc(nx2c), br(nx2r))
        iy2 = jnp.minimum(bc(ny2c), br(ny2r))
        inter = (jnp.clip(ix2 - ix1, 0.0, None) *
                 jnp.clip(iy2 - iy1, 0.0, None))
        union = bc(areac) + br(arear) - inter
        iou_scr[sl, :] = inter / jnp.maximum(union, 1e-8)

    ar = lax.broadcasted_iota(i32, (1, _KP), 1)
    kp0 = _t(buf[:, 6:7])

    def nms_body(i, kp):
        row = iou_scr[pl.ds(i, 1), :]
        cur = jnp.max(jnp.where(ar == i, kp, 0.0))
        sup = (row > _NMS_THRESH) & (ar > i) & (cur > 0.5)
        return jnp.where(sup, 0.0, kp)

    kp = lax.fori_loop(0, _PRE_NMS_TOP_N, nms_body, kp0)

    sc2 = _t(buf[:, 4:5]) * kp

    def sel_body(t, sc):
        m = jnp.max(sc)
        j = jnp.min(jnp.where(sc == m, ar, big))
        drow = det_scr[pl.ds(j, 1), :]
        orow = jnp.concatenate(
            [drow[:, 0:4], m.reshape(1, 1), drow[:, 5:6],
             jnp.zeros((1, 2), f32)], axis=1)
        out_scr[pl.ds(t, 1), :] = orow
        return jnp.where(ar == j, -1.0, sc)

    lax.fori_loop(0, _POST_TOP_N, sel_body, sc2)

    ob = out_scr[0:_POST_TOP_N, :]
    boxes_ref[0] = ob[:, 0:4]
    vals_ref[0] = _t(ob[:, 4:5])
    labels_ref[0] = _t(ob[:, 5:6]).astype(i32)


def kernel(locations, box_cls, box_regression, centerness, image_sizes):
    n, c, h, w = box_cls.shape
    hw = h * w
    rows = (hw * c) // 128
    cls_lin = jnp.transpose(box_cls, (0, 2, 3, 1)).reshape(n, rows, 128)
    ctr_hw = jnp.transpose(centerness, (0, 2, 3, 1)).reshape(n, hw, 1)
    ctr_lin = jnp.broadcast_to(ctr_hw, (n, hw, c)).reshape(n, rows, 128)
    reg_t = jnp.transpose(box_regression, (0, 2, 3, 1)).reshape(n, hw, 4)
    loc_reg = jnp.concatenate(
        [jnp.broadcast_to(locations[None], (n, hw, 2)), reg_t,
         jnp.zeros((n, hw, 2), jnp.float32)], axis=2)
    szf = image_sizes.astype(jnp.float32).reshape(n, 1, 2)

    boxes, vals, labels = pl.pallas_call(
        _fcos_kernel,
        grid=(n,),
        in_specs=[
            pl.BlockSpec((1, rows, 128), lambda i: (i, 0, 0)),
            pl.BlockSpec((1, rows, 128), lambda i: (i, 0, 0)),
            pl.BlockSpec((1, hw, 8), lambda i: (i, 0, 0)),
            pl.BlockSpec((1, 1, 2), lambda i: (i, 0, 0)),
        ],
        out_specs=[
            pl.BlockSpec((1, _POST_TOP_N, 4), lambda i: (i, 0, 0)),
            pl.BlockSpec((1, 1, _POST_TOP_N), lambda i: (i, 0, 0)),
            pl.BlockSpec((1, 1, _POST_TOP_N), lambda i: (i, 0, 0)),
        ],
        out_shape=[
            jax.ShapeDtypeStruct((n, _POST_TOP_N, 4), jnp.float32),
            jax.ShapeDtypeStruct((n, 1, _POST_TOP_N), jnp.float32),
            jax.ShapeDtypeStruct((n, 1, _POST_TOP_N), jnp.int32),
        ],
        compiler_params=pltpu.CompilerParams(
            vmem_limit_bytes=63 * 1024 * 1024),
        scratch_shapes=[
            pltpu.VMEM((_G0 * 8, 128), jnp.float32),
            pltpu.VMEM((_G0P, 128), jnp.float32),
            pltpu.VMEM((_G1P, 128), jnp.float32),
            pltpu.VMEM((_KP, 8), jnp.float32),
            pltpu.VMEM((_KP, _KP), jnp.float32),
            pltpu.VMEM((128, 8), jnp.float32),
        ],
    )(cls_lin, ctr_lin, loc_reg, szf)
    return boxes, vals.reshape(n, _POST_TOP_N), labels.reshape(n, _POST_TOP_N)


# speedup vs baseline: 3.0384x; 1.0903x over previous
import jax
import jax.numpy as jnp
from jax import lax
from jax.experimental import pallas as pl
from jax.experimental.pallas import tpu as pltpu

_PRE_NMS_THRESH = 0.05
_PRE_NMS_TOP_N = 1000
_NMS_THRESH = 0.6
_POST_TOP_N = 100
_NEG = -1e9

_N = 2
_HW = 15200
_C = 80
_ROWS = (_HW * _C) // 128
_G0 = 1188
_G0P = 1192
_G1 = 149
_G1P = 152
_G2 = 19
_KP = 1024


def _t(x):
    return lax.dot_general(
        jnp.ones((1, 1), jnp.float32), x, (((1,), (1,)), ((), ())),
        precision=lax.Precision.HIGHEST,
        preferred_element_type=jnp.float32)


def _group_max8(x, rows):
    r = x.reshape(rows, 8, 128)
    g = r[:, 0, :]
    for s in range(1, 8):
        g = jnp.maximum(g, r[:, s, :])
    return g


def _fcos_kernel(cls_ref, ctr_ref, lr_ref, sz_ref,
                 boxes_ref, vals_ref, labels_ref,
                 score_scr, g_scr, g2_scr, det_scr, iou_scr, out_scr):
    f32 = jnp.float32
    i32 = jnp.int32

    g3s = []
    pre_tops = []
    wm1s = []
    hm1s = []
    for img in range(_N):
        sig_cls = jax.nn.sigmoid(cls_ref[img])
        sig_ctr = jax.nn.sigmoid(ctr_ref[img])
        cand = sig_cls > _PRE_NMS_THRESH
        pre_tops.append(
            jnp.minimum(jnp.sum(cand.astype(i32)), _PRE_NMS_TOP_N))
        masked = jnp.where(cand, sig_cls * sig_ctr, _NEG)
        padded = jnp.concatenate(
            [masked, jnp.full((_G0 * 8 - _ROWS, 128), _NEG, f32)], axis=0)
        score_scr[img] = padded
        g0 = jnp.concatenate(
            [_group_max8(padded, _G0),
             jnp.full((_G0P - _G0, 128), _NEG, f32)], axis=0)
        g_scr[img] = g0
        g1 = jnp.concatenate(
            [_group_max8(g0, _G1),
             jnp.full((_G1P - _G1, 128), _NEG, f32)], axis=0)
        g2_scr[img] = g1
        g3s.append(_group_max8(g1, _G2))
        hm1s.append(sz_ref[img, 0, 0] - 1.0)
        wm1s.append(sz_ref[img, 0, 1] - 1.0)

    lane128 = lax.broadcasted_iota(i32, (1, 128), 1)
    r19 = lax.broadcasted_iota(i32, (_G2, 128), 0)
    r8 = lax.broadcasted_iota(i32, (8, 128), 0)
    t_iota = (lax.broadcasted_iota(i32, (8, 128), 0) * 128 +
              lax.broadcasted_iota(i32, (8, 128), 1))
    big = jnp.int32(2 ** 30)

    def extract_body(k, g3_pair):
        out = []
        for img in range(_N):
            g3 = g3_pair[img]
            m1 = jnp.max(g3)
            a3 = jnp.min(jnp.where(g3 == m1, r19, big))
            g2t = g2_scr[img, pl.ds(a3 * 8, 8), :]
            a2 = a3 * 8 + jnp.min(jnp.where(g2t == m1, r8, big))
            g1t = g_scr[img, pl.ds(a2 * 8, 8), :]
            gr = a2 * 8 + jnp.min(jnp.where(g1t == m1, r8, big))
            tile = score_scr[img, pl.ds(gr * 8, 8), :]
            p = jnp.min(jnp.where(tile == m1, t_iota, big))
            idx = gr * 1024 + p

            ntile = jnp.where(t_iota == p, _NEG, tile)
            score_scr[img, pl.ds(gr * 8, 8), :] = ntile
            g_scr[img, pl.ds(gr, 1), :] = jnp.max(ntile, axis=0,
                                                  keepdims=True)
            gslab = g_scr[img, pl.ds(a2 * 8, 8), :]
            g2_scr[img, pl.ds(a2, 1), :] = jnp.max(gslab, axis=0,
                                                   keepdims=True)
            g2slab = g2_scr[img, pl.ds(a3 * 8, 8), :]
            nrow3 = jnp.max(g2slab, axis=0, keepdims=True)
            g3 = jnp.where(r19 == a3, jnp.broadcast_to(nrow3, (_G2, 128)),
                           g3)

            li = idx // _C
            ci = (idx - li * _C + 1).astype(f32)
            lrow = lr_ref[img, pl.ds(li // 16, 1), :]
            m16 = li % 16

            def pick(comp):
                return jnp.sum(
                    jnp.where(lane128 == comp * 16 + m16, lrow, 0.0))

            lx = pick(0)
            ly = pick(1)
            x1 = jnp.clip(lx - pick(2), 0.0, wm1s[img])
            y1 = jnp.clip(ly - pick(3), 0.0, hm1s[img])
            x2 = jnp.clip(lx + pick(4), 0.0, wm1s[img])
            y2 = jnp.clip(ly + pick(5), 0.0, hm1s[img])
            keep = ((m1 > -1e8) & (k < pre_tops[img]) &
                    (x2 - x1 >= 0.0) & (y2 - y1 >= 0.0))
            kf = jnp.where(keep, 1.0, 0.0)
            scv = jnp.where(keep, m1, 0.0)
            row = jnp.concatenate(
                [x1.reshape(1, 1), y1.reshape(1, 1), x2.reshape(1, 1),
                 y2.reshape(1, 1), scv.reshape(1, 1), ci.reshape(1, 1),
                 kf.reshape(1, 1), jnp.zeros((1, 1), f32)], axis=1)
            det_scr[img, pl.ds(k, 1), :] = row
            out.append(g3)
        return tuple(out)

    lax.fori_loop(0, _PRE_NMS_TOP_N, extract_body, tuple(g3s))
    for img in range(_N):
        det_scr[img, _PRE_NMS_TOP_N:_KP, :] = jnp.zeros(
            (_KP - _PRE_NMS_TOP_N, 8), f32)

    kp0s = []
    sc0s = []
    for img in range(_N):
        buf = det_scr[img]
        off = buf[:, 5:6] * 4096.0
        nx1c = buf[:, 0:1] + off
        ny1c = buf[:, 1:2] + off
        nx2c = buf[:, 2:3] + off
        ny2c = buf[:, 3:4] + off
        areac = (jnp.clip(nx2c - nx1c, 0.0, None) *
                 jnp.clip(ny2c - ny1c, 0.0, None))
        nx1r = _t(nx1c)
        ny1r = _t(ny1c)
        nx2r = _t(nx2c)
        ny2r = _t(ny2c)
        arear = _t(areac)

        for s in range(_KP // 128):
            sl = slice(s * 128, (s + 1) * 128)
            bc = lambda q: jnp.broadcast_to(q[sl], (128, _KP))
            br = lambda q: jnp.broadcast_to(q, (128, _KP))
            ix1 = jnp.maximum(bc(nx1c), br(nx1r))
            iy1 = jnp.maximum(bc(ny1c), br(ny1r))
            ix2 = jnp.minimum(bc(nx2c), br(nx2r))
            iy2 = jnp.minimum(bc(ny2c), br(ny2r))
            inter = (jnp.clip(ix2 - ix1, 0.0, None) *
                     jnp.clip(iy2 - iy1, 0.0, None))
            union = bc(areac) + br(arear) - inter
            iou_scr[img, sl, :] = inter / jnp.maximum(union, 1e-8)

        kp0s.append(_t(buf[:, 6:7]))
        sc0s.append(_t(buf[:, 4:5]))

    ar = lax.broadcasted_iota(i32, (1, _KP), 1)

    def nms_body(i, kp_pair):
        out = []
        for img in range(_N):
            kp = kp_pair[img]
            row = iou_scr[img, pl.ds(i, 1), :]
            cur = jnp.max(jnp.where(ar == i, kp, 0.0))
            sup = (row > _NMS_THRESH) & (ar > i) & (cur > 0.5)
            out.append(jnp.where(sup, 0.0, kp))
        return tuple(out)

    kps = lax.fori_loop(0, _PRE_NMS_TOP_N, nms_body, tuple(kp0s))

    def sel_body(t, sc_pair):
        out = []
        for img in range(_N):
            sc = sc_pair[img]
            m = jnp.max(sc)
            j = jnp.min(jnp.where(sc == m, ar, big))
            drow = det_scr[img, pl.ds(j, 1), :]
            orow = jnp.concatenate(
                [drow[:, 0:4], m.reshape(1, 1), drow[:, 5:6],
                 jnp.zeros((1, 2), f32)], axis=1)
            out_scr[img, pl.ds(t, 1), :] = orow
            out.append(jnp.where(ar == j, -1.0, sc))
        return tuple(out)

    lax.fori_loop(0, _POST_TOP_N, sel_body,
                  tuple(s * k for s, k in zip(sc0s, kps)))

    for img in range(_N):
        ob = out_scr[img, 0:_POST_TOP_N, :]
        boxes_ref[img] = ob[:, 0:4]
        vals_ref[img] = _t(ob[:, 4:5])
        labels_ref[img] = _t(ob[:, 5:6]).astype(i32)


def kernel(locations, box_cls, box_regression, centerness, image_sizes):
    n, c, h, w = box_cls.shape
    hw = h * w
    rows = (hw * c) // 128
    cls_lin = jnp.transpose(box_cls, (0, 2, 3, 1)).reshape(n, rows, 128)
    ctr_hw = jnp.transpose(centerness, (0, 2, 3, 1)).reshape(n, hw, 1)
    ctr_lin = jnp.broadcast_to(ctr_hw, (n, hw, c)).reshape(n, rows, 128)
    reg_t = jnp.transpose(box_regression, (0, 2, 3, 1)).reshape(n, hw, 4)
    loc_reg = jnp.concatenate(
        [jnp.broadcast_to(locations[None], (n, hw, 2)), reg_t,
         jnp.zeros((n, hw, 2), jnp.float32)], axis=2)
    lr_packed = loc_reg.reshape(n, hw // 16, 16, 8).transpose(
        0, 1, 3, 2).reshape(n, hw // 16, 128)
    szf = image_sizes.astype(jnp.float32).reshape(n, 1, 2)

    boxes, vals, labels = pl.pallas_call(
        _fcos_kernel,
        out_shape=[
            jax.ShapeDtypeStruct((n, _POST_TOP_N, 4), jnp.float32),
            jax.ShapeDtypeStruct((n, 1, _POST_TOP_N), jnp.float32),
            jax.ShapeDtypeStruct((n, 1, _POST_TOP_N), jnp.int32),
        ],
        compiler_params=pltpu.CompilerParams(
            vmem_limit_bytes=63 * 1024 * 1024),
        scratch_shapes=[
            pltpu.VMEM((_N, _G0 * 8, 128), jnp.float32),
            pltpu.VMEM((_N, _G0P, 128), jnp.float32),
            pltpu.VMEM((_N, _G1P, 128), jnp.float32),
            pltpu.VMEM((_N, _KP, 8), jnp.float32),
            pltpu.VMEM((_N, _KP, _KP), jnp.float32),
            pltpu.VMEM((_N, 128, 8), jnp.float32),
        ],
    )(cls_lin, ctr_lin, lr_packed, szf)
    return boxes, vals.reshape(n, _POST_TOP_N), labels.reshape(n, _POST_TOP_N)
